# baseline (speedup 1.0000x reference)
import functools

import jax
import jax.numpy as jnp
from jax import lax
from jax.experimental import pallas as pl
from jax.experimental.pallas import tpu as pltpu
from jax.experimental.pallas import tpu_sc as plsc


def _pooler_kernel(B, S, D):
    mesh = plsc.VectorSubcoreMesh(core_axis_name="c", subcore_axis_name="s")

    @functools.partial(
        pl.kernel,
        mesh=mesh,
        out_type=jax.ShapeDtypeStruct((B, D), jnp.float32),
        scratch_types=[
            pltpu.VMEM((S + 16,), jnp.int32),
            pltpu.VMEM((D,), jnp.float32),
        ],
    )
    def k(hs_hbm, mask_hbm, out_hbm, mask_v, row_v):
        num_cores = 2
        wid = lax.axis_index("s") * num_cores + lax.axis_index("c")

        @pl.when(wid < B)
        def _():
            b = wid
            pltpu.sync_copy(mask_hbm.at[b], mask_v.at[pl.ds(0, S)])
            mask_v[pl.ds(S, 16)] = jnp.zeros((16,), jnp.int32)

            lo = jnp.int32(0)
            sz = S // 2
            while sz >= 1:
                probe = mask_v[pl.ds(lo + sz, 16)][0]
                lo = lo + jnp.where(probe == 1, jnp.int32(sz), jnp.int32(0))
                sz //= 2
            idx = lo

            pltpu.sync_copy(hs_hbm.at[b, idx], row_v)
            pltpu.sync_copy(row_v, out_hbm.at[b])

    return k


def kernel(hidden_state, attention_mask):
    B, S, D = hidden_state.shape
    return _pooler_kernel(B, S, D)(hidden_state, attention_mask)


# speedup vs baseline: 1.0377x; 1.0377x over previous
import functools

import jax
import jax.numpy as jnp
from jax import lax
from jax.experimental import pallas as pl
from jax.experimental.pallas import tpu as pltpu
from jax.experimental.pallas import tpu_sc as plsc


def _pooler_kernel(B, S, D):
    mesh = plsc.VectorSubcoreMesh(
        core_axis_name="c", subcore_axis_name="s", num_cores=1
    )

    @functools.partial(
        pl.kernel,
        mesh=mesh,
        out_type=jax.ShapeDtypeStruct((B, D), jnp.float32),
        scratch_types=[
            pltpu.VMEM((S + 16,), jnp.int32),
        ],
    )
    def k(hs_hbm, mask_hbm, out_hbm, mask_v):
        wid = lax.axis_index("s")

        @pl.when(wid < B)
        def _():
            b = wid
            pltpu.sync_copy(mask_hbm.at[b], mask_v.at[pl.ds(0, S)])
            mask_v[pl.ds(S, 16)] = jnp.zeros((16,), jnp.int32)

            lo = jnp.int32(0)
            sz = S // 2
            while sz >= 1:
                probe = mask_v[pl.ds(lo + sz, 16)][0]
                lo = lo + jnp.where(probe == 1, jnp.int32(sz), jnp.int32(0))
                sz //= 2
            idx = lo

            pltpu.sync_copy(hs_hbm.at[b, idx], out_hbm.at[b])

    return k


def kernel(hidden_state, attention_mask):
    B, S, D = hidden_state.shape
    return _pooler_kernel(B, S, D)(hidden_state, attention_mask)


# speedup vs baseline: 6.6284x; 6.3874x over previous
import jax
import jax.numpy as jnp
from jax.experimental import pallas as pl
from jax.experimental.pallas import tpu as pltpu


def _body(B):
    def body(mask_ref, hs_ref, out_ref, sem):
        for b in range(B):
            total = jnp.sum(mask_ref[b, :])
            idx = jnp.maximum(total - 1, 0)
            pltpu.make_async_copy(
                hs_ref.at[b, idx], out_ref.at[b], sem.at[b]
            ).start()
        for b in range(B):
            pltpu.make_async_copy(
                hs_ref.at[0, 0], out_ref.at[b], sem.at[b]
            ).wait()

    return body


def kernel(hidden_state, attention_mask):
    B, S, D = hidden_state.shape
    return pl.pallas_call(
        _body(B),
        out_shape=jax.ShapeDtypeStruct((B, D), jnp.float32),
        in_specs=[
            pl.BlockSpec(memory_space=pltpu.VMEM),
            pl.BlockSpec(memory_space=pltpu.MemorySpace.HBM),
        ],
        out_specs=pl.BlockSpec(memory_space=pltpu.MemorySpace.HBM),
        scratch_shapes=[pltpu.SemaphoreType.DMA((B,))],
    )(attention_mask, hidden_state)


# speedup vs baseline: 6.7471x; 1.0179x over previous
import jax
import jax.numpy as jnp
from jax.experimental import pallas as pl
from jax.experimental.pallas import tpu as pltpu


def _body(B, S):
    def body(mask_hbm, hs_ref, out_ref, mask_v, copy_sem, row_sem):
        for b in range(B):
            pltpu.make_async_copy(
                mask_hbm.at[b], mask_v.at[b], copy_sem.at[b]
            ).start()
        for b in range(B):
            pltpu.make_async_copy(
                mask_hbm.at[b], mask_v.at[b], copy_sem.at[b]
            ).wait()
            total = jnp.sum(mask_v[b, :])
            idx = jnp.maximum(total - 1, 0)
            pltpu.make_async_copy(
                hs_ref.at[b, idx], out_ref.at[b], row_sem.at[b]
            ).start()
        for b in range(B):
            pltpu.make_async_copy(
                hs_ref.at[0, 0], out_ref.at[b], row_sem.at[b]
            ).wait()

    return body


def kernel(hidden_state, attention_mask):
    B, S, D = hidden_state.shape
    return pl.pallas_call(
        _body(B, S),
        out_shape=jax.ShapeDtypeStruct((B, D), jnp.float32),
        in_specs=[
            pl.BlockSpec(memory_space=pltpu.MemorySpace.HBM),
            pl.BlockSpec(memory_space=pltpu.MemorySpace.HBM),
        ],
        out_specs=pl.BlockSpec(memory_space=pltpu.MemorySpace.HBM),
        scratch_shapes=[
            pltpu.VMEM((B, S), jnp.int32),
            pltpu.SemaphoreType.DMA((B,)),
            pltpu.SemaphoreType.DMA((B,)),
        ],
    )(attention_mask, hidden_state)
